# baseline (speedup 1.0000x reference)
import functools

import jax
import jax.numpy as jnp
from jax import lax
from jax.experimental import pallas as pl
from jax.experimental.pallas import tpu as pltpu
from jax.experimental.pallas import tpu_sc as plsc

_BATCH = 4
_SEQ = 4096
_D = 1024

_NC = 2
_NS = 16
_NW = _NC * _NS

_SROWS = _SEQ // _NW
_CH = 32
_NCHUNK = _SROWS // _CH
_GRP = _D // 128


def _sc_body(x_hbm, p_hbm, o_hbm,
             posbuf, io0, io1,
             isem0, isem1, osem0, osem1):
    wid = lax.axis_index("s") * _NC + lax.axis_index("c")
    base = wid * _SROWS

    ios = (io0, io1)
    isems = (isem0, isem1)
    osems = (osem0, osem1)

    pltpu.async_copy(x_hbm.at[0, pl.ds(base, _CH)], io0, isem0)

    def chunk(j, carry):
        row0 = base + j * _CH
        pltpu.sync_copy(p_hbm.at[pl.ds(row0, _CH)], posbuf)
        for b in range(_BATCH):
            slot = b % 2
            other = 1 - slot
            if b == 0:
                @pl.when(j > 0)
                def _():
                    pltpu.make_async_copy(
                        ios[other], o_hbm.at[0, pl.ds(0, _CH)],
                        osems[other]).wait()
            else:
                pltpu.make_async_copy(
                    ios[other], o_hbm.at[0, pl.ds(0, _CH)],
                    osems[other]).wait()
            if b < _BATCH - 1:
                pltpu.async_copy(
                    x_hbm.at[b + 1, pl.ds(row0, _CH)],
                    ios[other], isems[other])
            else:
                @pl.when(j + 1 < _NCHUNK)
                def _():
                    pltpu.async_copy(
                        x_hbm.at[0, pl.ds(row0 + _CH, _CH)],
                        ios[other], isems[other])
            pltpu.make_async_copy(
                x_hbm.at[0, pl.ds(0, _CH)], ios[slot], isems[slot]).wait()

            def add_body(i, c):
                r = i >> 3
                c0 = (i & 7) * 128
                for u in range(8):
                    sl = pl.ds(c0 + u * 16, 16)
                    ios[slot][r, sl] = ios[slot][r, sl] + posbuf[r, sl]
                return c

            lax.fori_loop(0, _CH * _GRP, add_body, 0)
            pltpu.async_copy(
                ios[slot], o_hbm.at[b, pl.ds(row0, _CH)], osems[slot])
        return carry

    lax.fori_loop(0, _NCHUNK, chunk, 0)
    pltpu.make_async_copy(
        ios[(_BATCH - 1) % 2], o_hbm.at[0, pl.ds(0, _CH)],
        osems[(_BATCH - 1) % 2]).wait()


_sc_add = functools.partial(
    pl.kernel,
    mesh=plsc.VectorSubcoreMesh(core_axis_name="c", subcore_axis_name="s"),
    out_type=jax.ShapeDtypeStruct((_BATCH, _SEQ, _D), jnp.float32),
    scratch_types=[
        pltpu.VMEM((_CH, _D), jnp.float32),
        pltpu.VMEM((_CH, _D), jnp.float32),
        pltpu.VMEM((_CH, _D), jnp.float32),
        pltpu.SemaphoreType.DMA,
        pltpu.SemaphoreType.DMA,
        pltpu.SemaphoreType.DMA,
        pltpu.SemaphoreType.DMA,
    ],
)(_sc_body)


def kernel(inputs, pos_table):
    return _sc_add(inputs, pos_table)


# speedup vs baseline: 1.0635x; 1.0635x over previous
import functools

import jax
import jax.numpy as jnp
from jax import lax
from jax.experimental import pallas as pl
from jax.experimental.pallas import tpu as pltpu
from jax.experimental.pallas import tpu_sc as plsc

_BATCH = 4
_SEQ = 4096
_D = 1024

_NC = 2
_NS = 16
_NW = _NC * _NS

_SROWS = _SEQ // _NW
_CH = 16
_NCHUNK = _SROWS // _CH
_GRP = _D // 128


def _sc_body(x_hbm, p_hbm, o_hbm,
             posbuf, in0, in1, out0, out1,
             isem0, isem1, osem0, osem1):
    wid = lax.axis_index("s") * _NC + lax.axis_index("c")
    base = wid * _SROWS

    ins = (in0, in1)
    outs = (out0, out1)
    isems = (isem0, isem1)
    osems = (osem0, osem1)

    pltpu.async_copy(x_hbm.at[pl.ds(base, _CH)], in0, isem0)

    def chunk(j, carry):
        row0 = base + j * _CH
        pltpu.sync_copy(p_hbm.at[pl.ds(row0, _CH)], posbuf)
        for b in range(_BATCH):
            slot = b % 2
            if b < _BATCH - 1:
                pltpu.async_copy(
                    x_hbm.at[pl.ds((b + 1) * _SEQ + row0, _CH)],
                    ins[1 - slot], isems[1 - slot])
            else:
                @pl.when(j + 1 < _NCHUNK)
                def _():
                    pltpu.async_copy(
                        x_hbm.at[pl.ds(row0 + _CH, _CH)],
                        ins[1 - slot], isems[1 - slot])
            pltpu.make_async_copy(
                x_hbm.at[pl.ds(0, _CH)], ins[slot], isems[slot]).wait()
            if b >= 2:
                pltpu.make_async_copy(
                    outs[slot], o_hbm.at[pl.ds(0, _CH)], osems[slot]).wait()
            else:
                @pl.when(j > 0)
                def _():
                    pltpu.make_async_copy(
                        outs[slot], o_hbm.at[pl.ds(0, _CH)],
                        osems[slot]).wait()

            def add_body(i, c):
                r = i >> 3
                c0 = (i & 7) * 128
                for u in range(8):
                    sl = pl.ds(c0 + u * 16, 16)
                    outs[slot][r, sl] = ins[slot][r, sl] + posbuf[r, sl]
                return c

            lax.fori_loop(0, _CH * _GRP, add_body, 0)
            pltpu.async_copy(
                outs[slot], o_hbm.at[pl.ds(b * _SEQ + row0, _CH)],
                osems[slot])
        return carry

    lax.fori_loop(0, _NCHUNK, chunk, 0)
    pltpu.make_async_copy(out0, o_hbm.at[pl.ds(0, _CH)], osem0).wait()
    pltpu.make_async_copy(out1, o_hbm.at[pl.ds(0, _CH)], osem1).wait()


_sc_add = functools.partial(
    pl.kernel,
    mesh=plsc.VectorSubcoreMesh(core_axis_name="c", subcore_axis_name="s"),
    out_type=jax.ShapeDtypeStruct((_BATCH * _SEQ, _D), jnp.float32),
    scratch_types=[
        pltpu.VMEM((_CH, _D), jnp.float32),
        pltpu.VMEM((_CH, _D), jnp.float32),
        pltpu.VMEM((_CH, _D), jnp.float32),
        pltpu.VMEM((_CH, _D), jnp.float32),
        pltpu.VMEM((_CH, _D), jnp.float32),
        pltpu.SemaphoreType.DMA,
        pltpu.SemaphoreType.DMA,
        pltpu.SemaphoreType.DMA,
        pltpu.SemaphoreType.DMA,
    ],
)(_sc_body)


def kernel(inputs, pos_table):
    out = _sc_add(inputs.reshape(_BATCH * _SEQ, _D), pos_table)
    return out.reshape(inputs.shape)
